# Initial kernel scaffold; baseline (speedup 1.0000x reference)
import jax
import jax.numpy as jnp
from jax.experimental import pallas as pl


def kernel(ref_bxyz, query_bxyz, e_ref, e_query, kernel_pos):
    raise NotImplementedError("write your pallas kernel here")


# speedup vs baseline: 14.8581x; 14.8581x over previous
import functools

import jax
import jax.numpy as jnp
from jax import lax
from jax.experimental import pallas as pl
from jax.experimental.pallas import tpu as pltpu
from jax.experimental.pallas import tpu_sc as plsc

_NC = 2
_NS = 16
_NW = _NC * _NS
_L = 16


def _make_sc_assign(E):
    per = E // _NW
    C = 5000 if per % 5000 == 0 else per
    n_chunks = per // C
    n_vec = (C + _L - 1) // _L
    c_pad = n_vec * _L

    mesh = plsc.VectorSubcoreMesh(core_axis_name="c", subcore_axis_name="s")

    idx_buf = pltpu.VMEM((C,), jnp.int32)
    data_buf = pltpu.VMEM((c_pad,), jnp.float32)
    out_buf = pltpu.VMEM((c_pad,), jnp.int32)

    @functools.partial(
        pl.kernel,
        mesh=mesh,
        out_type=jax.ShapeDtypeStruct((E,), jnp.int32),
        scratch_types=[
            idx_buf, idx_buf,
            idx_buf, idx_buf,
            data_buf, data_buf, data_buf,
            data_buf, data_buf, data_buf,
            data_buf, data_buf, data_buf,
            data_buf, data_buf, data_buf,
            out_buf, out_buf,
            pltpu.VMEM((6 * _L,), jnp.float32),
            pltpu.SemaphoreType.DMA,
            pltpu.SemaphoreType.DMA,
            pltpu.SemaphoreType.DMA,
            pltpu.SemaphoreType.DMA,
        ],
    )
    def sc_assign(rx_hbm, ry_hbm, rz_hbm, qx_hbm, qy_hbm, qz_hbm,
                  eref_hbm, equery_hbm, kp_hbm, out_hbm,
                  er0, er1, eq0, eq1,
                  rx0, ry0, rz0, rx1, ry1, rz1,
                  qx0, qy0, qz0, qx1, qy1, qz1,
                  out0, out1, kp_v,
                  gsem0, gsem1, osem0, osem1):
        wid = (lax.axis_index("s").astype(jnp.int32) * jnp.int32(_NC)
               + lax.axis_index("c").astype(jnp.int32))
        tile_base = wid * jnp.int32(per)

        pltpu.sync_copy(kp_hbm, kp_v)
        hxv = kp_v[pl.ds(0, _L)]
        hyv = kp_v[pl.ds(_L, _L)]
        hzv = kp_v[pl.ds(2 * _L, _L)]
        nhxv = kp_v[pl.ds(3 * _L, _L)]
        nhyv = kp_v[pl.ds(4 * _L, _L)]
        nhzv = kp_v[pl.ds(5 * _L, _L)]
        nine = jnp.full((_L,), 9, jnp.int32)
        three = jnp.full((_L,), 3, jnp.int32)
        one = jnp.full((_L,), 1, jnp.int32)
        zero = jnp.full((_L,), 0, jnp.int32)

        bufs = [
            (er0, eq0, rx0, ry0, rz0, qx0, qy0, qz0, out0, gsem0, osem0),
            (er1, eq1, rx1, ry1, rz1, qx1, qy1, qz1, out1, gsem1, osem1),
        ]

        def stage(j):
            er, eq, rx, ry, rz, qx, qy, qz, _, gsem, _ = bufs[j % 2]
            base_e = tile_base + jnp.int32(j * C)
            pltpu.sync_copy(eref_hbm.at[pl.ds(base_e, C)], er)
            pltpu.sync_copy(equery_hbm.at[pl.ds(base_e, C)], eq)
            return [
                pltpu.async_copy(rx_hbm.at[er], rx.at[pl.ds(0, C)], gsem),
                pltpu.async_copy(ry_hbm.at[er], ry.at[pl.ds(0, C)], gsem),
                pltpu.async_copy(rz_hbm.at[er], rz.at[pl.ds(0, C)], gsem),
                pltpu.async_copy(qx_hbm.at[eq], qx.at[pl.ds(0, C)], gsem),
                pltpu.async_copy(qy_hbm.at[eq], qy.at[pl.ds(0, C)], gsem),
                pltpu.async_copy(qz_hbm.at[eq], qz.at[pl.ds(0, C)], gsem),
            ]

        pending = {0: stage(0)}
        out_cp = {}
        for j in range(n_chunks):
            if j + 1 < n_chunks:
                pending[j + 1] = stage(j + 1)
            for cp in pending.pop(j):
                cp.wait()
            _, _, rx, ry, rz, qx, qy, qz, out_v, _, osem = bufs[j % 2]
            if j - 2 in out_cp:
                out_cp.pop(j - 2).wait()

            def body(i, carry):
                sl = pl.ds(i * jnp.int32(_L), _L)
                tx = rx[sl] - qx[sl]
                ty = ry[sl] - qy[sl]
                tz = rz[sl] - qz[sl]
                out_v[sl] = (jnp.where(tx > nhxv, nine, zero)
                             + jnp.where(tx > hxv, nine, zero)
                             + jnp.where(ty > nhyv, three, zero)
                             + jnp.where(ty > hyv, three, zero)
                             + jnp.where(tz > nhzv, one, zero)
                             + jnp.where(tz > hzv, one, zero))
                return carry

            lax.fori_loop(jnp.int32(0), jnp.int32(n_vec), body, 0,
                          unroll=False)
            base_e = tile_base + jnp.int32(j * C)
            out_cp[j] = pltpu.async_copy(out_v.at[pl.ds(0, C)],
                                         out_hbm.at[pl.ds(base_e, C)], osem)
        for cp in out_cp.values():
            cp.wait()

    return sc_assign


def kernel(ref_bxyz, query_bxyz, e_ref, e_query, kernel_pos):
    E = e_ref.shape[0]
    er = e_ref.astype(jnp.int32)
    eq = e_query.astype(jnp.int32)
    rx, ry, rz = ref_bxyz[:, 1], ref_bxyz[:, 2], ref_bxyz[:, 3]
    qx, qy, qz = query_bxyz[:, 1], query_bxyz[:, 2], query_bxyz[:, 3]
    h = kernel_pos[26, :].astype(jnp.float32) * jnp.float32(0.5)
    kp_pad = jnp.concatenate([jnp.repeat(h, _L), jnp.repeat(-h, _L)])
    out32 = _make_sc_assign(E)(rx, ry, rz, qx, qy, qz, er, eq, kp_pad)
    return out32.astype(jnp.int64)
